# Initial kernel scaffold; baseline (speedup 1.0000x reference)
import jax
import jax.numpy as jnp
from jax.experimental import pallas as pl


def kernel(cat_inputs, num_inputs, T, Wn, bn, W1, b1, W2, b2, gamma, beta):
    raise NotImplementedError("write your pallas kernel here")


# speedup vs baseline: 3.7996x; 3.7996x over previous
import functools

import jax
import jax.numpy as jnp
from jax import lax
from jax.experimental import pallas as pl
from jax.experimental.pallas import tpu as pltpu
from jax.experimental.pallas import tpu_sc as plsc

_B = 16384
_NC = 26
_NN = 13
_V = 100000
_D = 32
_H = 2 * _D
_NIDX = _B * _NC

_GW = 512
_RB_CAT = 4096
_TILE_N = 256
_RB_NUM = _TILE_N * _NN

_SQRT2 = 1.4142135623730951


_NW = 32
_BPW = _NIDX // _NW
_CH = 512
_NCH = _BPW // _CH


def _sc_gather(t_flat, idx):
    mesh = plsc.VectorSubcoreMesh(core_axis_name="core", subcore_axis_name="subcore")

    @functools.partial(
        pl.kernel,
        out_type=jax.ShapeDtypeStruct((_NIDX, _D), jnp.float32),
        mesh=mesh,
        scratch_types=[pltpu.VMEM((_CH,), jnp.int32),
                       pltpu.VMEM((_CH, _D), jnp.float32),
                       pltpu.SemaphoreType.DMA],
        compiler_params=pltpu.CompilerParams(use_tc_tiling_on_sc=False),
    )
    def k(x_hbm, i_hbm, o_hbm, idx_v, rows_v, sem):
        wid = lax.axis_index("subcore") * 2 + lax.axis_index("core")
        base = wid * _BPW

        @pl.loop(0, _NCH)
        def _(c):
            off = base + c * _CH
            pltpu.sync_copy(i_hbm.at[pl.ds(off, _CH)], idx_v)
            pltpu.async_copy(x_hbm.at[idx_v], rows_v, sem).wait()
            pltpu.sync_copy(rows_v, o_hbm.at[pl.ds(off, _CH)])

    return k(t_flat, idx)


def _mlp(x, w1, b1, w2, b2, gamma, beta):
    h = jnp.dot(x, w1, precision=lax.Precision.HIGHEST,
                preferred_element_type=jnp.float32) + b1
    h = 0.5 * h * (1.0 + lax.erf(h / _SQRT2))
    h = jnp.dot(h, w2, precision=lax.Precision.HIGHEST,
                preferred_element_type=jnp.float32) + b2
    mu = jnp.mean(h, axis=1, keepdims=True)
    c = h - mu
    var = jnp.mean(c * c, axis=1, keepdims=True)
    return c * lax.rsqrt(var + 1e-5) * gamma + beta


def _cat_body(x_ref, w1_ref, b1_ref, w2_ref, b2_ref, g_ref, be_ref, o_ref):
    o_ref[...] = _mlp(x_ref[...], w1_ref[...], b1_ref[...], w2_ref[...],
                      b2_ref[...], g_ref[...], be_ref[...])


def _num_body(v_ref, wn_ref, bn_ref, w1_ref, b1_ref, w2_ref, b2_ref,
              g_ref, be_ref, o_ref):
    x = jnp.broadcast_to(v_ref[...], (_RB_NUM, _D)) * wn_ref[...] + bn_ref[...]
    o_ref[...] = _mlp(x, w1_ref[...], b1_ref[...], w2_ref[...],
                      b2_ref[...], g_ref[...], be_ref[...])


def _full(shape):
    return pl.BlockSpec(shape, lambda i: (0, 0))


def kernel(cat_inputs, num_inputs, T, Wn, bn, W1, b1, W2, b2, gamma, beta):
    t_flat = T.reshape(_NC * _V, _D)
    idx = (cat_inputs.astype(jnp.int32)
           + (jnp.arange(_NC, dtype=jnp.int32) * _V)[None, :]).reshape(_NIDX)

    cat_rows = _sc_gather(t_flat, idx)

    b1r = b1.reshape(1, _H)
    b2r = b2.reshape(1, _D)
    gr = gamma.reshape(1, _D)
    ber = beta.reshape(1, _D)

    w_specs = [_full((_D, _H)), _full((1, _H)), _full((_H, _D)),
               _full((1, _D)), _full((1, _D)), _full((1, _D))]

    catm = pl.pallas_call(
        _cat_body,
        grid=(_NIDX // _RB_CAT,),
        in_specs=[pl.BlockSpec((_RB_CAT, _D), lambda i: (i, 0))] + w_specs,
        out_specs=pl.BlockSpec((_RB_CAT, _D), lambda i: (i, 0)),
        out_shape=jax.ShapeDtypeStruct((_NIDX, _D), jnp.float32),
    )(cat_rows, W1, b1r, W2, b2r, gr, ber)

    num_flat = num_inputs.reshape(_B * _NN, 1)
    wn_tile = jnp.tile(Wn, (_TILE_N, 1))
    bn_tile = jnp.tile(bn, (_TILE_N, 1))

    numm = pl.pallas_call(
        _num_body,
        grid=(_B * _NN // _RB_NUM,),
        in_specs=[pl.BlockSpec((_RB_NUM, 1), lambda i: (i, 0)),
                  _full((_RB_NUM, _D)), _full((_RB_NUM, _D))] + w_specs,
        out_specs=pl.BlockSpec((_RB_NUM, _D), lambda i: (i, 0)),
        out_shape=jax.ShapeDtypeStruct((_B * _NN, _D), jnp.float32),
    )(num_flat, wn_tile, bn_tile, W1, b1r, W2, b2r, gr, ber)

    return jnp.concatenate([catm.reshape(_B, _NC, _D),
                            numm.reshape(_B, _NN, _D)], axis=1)


# speedup vs baseline: 6.5143x; 1.7145x over previous
import functools

import jax
import jax.numpy as jnp
from jax import lax
from jax.experimental import pallas as pl
from jax.experimental.pallas import tpu as pltpu
from jax.experimental.pallas import tpu_sc as plsc

_B = 16384
_NC = 26
_NN = 13
_V = 100000
_D = 32
_H = 2 * _D
_NIDX = _B * _NC
_PK = 128 // _D

_NW = 32
_BPW = _NIDX // _NW
_CH = 512
_NCH = _BPW // _CH

_CAT_PROWS = _NIDX // _PK
_NUM_PROWS = _B * _NN // _PK
_RB_CAT = 2048
_RB_NUM = 1664

_SQRT2 = 1.4142135623730951


def _sc_gather(t_flat, idx):
    mesh = plsc.VectorSubcoreMesh(core_axis_name="core", subcore_axis_name="subcore")

    @functools.partial(
        pl.kernel,
        out_type=jax.ShapeDtypeStruct((_NIDX, _D), jnp.float32),
        mesh=mesh,
        scratch_types=[pltpu.VMEM((_CH,), jnp.int32),
                       pltpu.VMEM((_CH, _D), jnp.float32),
                       pltpu.SemaphoreType.DMA],
        compiler_params=pltpu.CompilerParams(use_tc_tiling_on_sc=False),
    )
    def k(x_hbm, i_hbm, o_hbm, idx_v, rows_v, sem):
        wid = lax.axis_index("subcore") * 2 + lax.axis_index("core")
        base = wid * _BPW

        @pl.loop(0, _NCH)
        def _(c):
            off = base + c * _CH
            pltpu.sync_copy(i_hbm.at[pl.ds(off, _CH)], idx_v)
            pltpu.async_copy(x_hbm.at[idx_v], rows_v, sem).wait()
            pltpu.sync_copy(rows_v, o_hbm.at[pl.ds(off, _CH)])

    return k(t_flat, idx)


def _mlp_packed(x, w1, b1, w2, b2, g, be, m):
    h = jnp.dot(x, w1, preferred_element_type=jnp.float32) + b1
    h = 0.5 * h * (1.0 + lax.erf(h / _SQRT2))
    y = jnp.dot(h, w2, preferred_element_type=jnp.float32) + b2
    mu = jnp.dot(y, m, preferred_element_type=jnp.float32)
    c = y - mu
    var = jnp.dot(c * c, m, preferred_element_type=jnp.float32)
    return c * lax.rsqrt(var + 1e-5) * g + be


def _cat_body(x_ref, w1_ref, b1_ref, w2_ref, b2_ref, g_ref, be_ref, m_ref,
              o_ref):
    o_ref[...] = _mlp_packed(x_ref[...], w1_ref[...], b1_ref[...], w2_ref[...],
                             b2_ref[...], g_ref[...], be_ref[...], m_ref[...])


def _num_body(v_ref, e_ref, wn_ref, bn_ref, w1_ref, b1_ref, w2_ref, b2_ref,
              g_ref, be_ref, m_ref, o_ref):
    v = jnp.dot(v_ref[...], e_ref[...], precision=lax.Precision.HIGHEST,
                preferred_element_type=jnp.float32)
    x = v * wn_ref[...] + bn_ref[...]
    o_ref[...] = _mlp_packed(x, w1_ref[...], b1_ref[...], w2_ref[...],
                             b2_ref[...], g_ref[...], be_ref[...], m_ref[...])


def _full(shape):
    return pl.BlockSpec(shape, lambda i: tuple(0 for _ in shape))


def kernel(cat_inputs, num_inputs, T, Wn, bn, W1, b1, W2, b2, gamma, beta):
    t_flat = T.reshape(_NC * _V, _D)
    idx = (cat_inputs.astype(jnp.int32)
           + (jnp.arange(_NC, dtype=jnp.int32) * _V)[None, :]).reshape(_NIDX)

    cat_rows = _sc_gather(t_flat, idx)
    cat_p = cat_rows.reshape(_CAT_PROWS, 128)

    eye4 = jnp.eye(_PK, dtype=jnp.float32)
    w1bd = jnp.kron(eye4, W1)
    w2bd = jnp.kron(eye4, W2)
    mavg = jnp.kron(eye4, jnp.full((_D, _D), 1.0 / _D, jnp.float32))
    b1t = jnp.tile(b1, _PK).reshape(1, _PK * _H)
    b2t = jnp.tile(b2, _PK).reshape(1, 128)
    gt = jnp.tile(gamma, _PK).reshape(1, 128)
    bet = jnp.tile(beta, _PK).reshape(1, 128)

    w_specs = [_full((128, _PK * _H)), _full((1, _PK * _H)),
               _full((_PK * _H, 128)), _full((1, 128)), _full((1, 128)),
               _full((1, 128)), _full((128, 128))]

    catm = pl.pallas_call(
        _cat_body,
        grid=(_CAT_PROWS // _RB_CAT,),
        in_specs=[pl.BlockSpec((_RB_CAT, 128), lambda i: (i, 0))] + w_specs,
        out_specs=pl.BlockSpec((_RB_CAT, 128), lambda i: (i, 0)),
        out_shape=jax.ShapeDtypeStruct((_CAT_PROWS, 128), jnp.float32),
    )(cat_p, w1bd, b1t, w2bd, b2t, gt, bet, mavg)

    num4 = num_inputs.reshape(_NUM_PROWS, _PK)
    e4 = jnp.kron(eye4, jnp.ones((1, _D), jnp.float32))
    wn_pat = jnp.tile(Wn.reshape(-1), _PK).reshape(_NN, 128)
    bn_pat = jnp.tile(bn.reshape(-1), _PK).reshape(_NN, 128)
    wn_tile = jnp.tile(wn_pat, (_RB_NUM // _NN, 1))
    bn_tile = jnp.tile(bn_pat, (_RB_NUM // _NN, 1))

    numm = pl.pallas_call(
        _num_body,
        grid=(_NUM_PROWS // _RB_NUM,),
        in_specs=[pl.BlockSpec((_RB_NUM, _PK), lambda i: (i, 0)),
                  _full((_PK, 128)),
                  _full((_RB_NUM, 128)), _full((_RB_NUM, 128))] + w_specs,
        out_specs=pl.BlockSpec((_RB_NUM, 128), lambda i: (i, 0)),
        out_shape=jax.ShapeDtypeStruct((_NUM_PROWS, 128), jnp.float32),
    )(num4, e4, wn_tile, bn_tile, w1bd, b1t, w2bd, b2t, gt, bet, mavg)

    return jnp.concatenate([catm.reshape(_B, _NC, _D),
                            numm.reshape(_B, _NN, _D)], axis=1)


# speedup vs baseline: 9.4343x; 1.4482x over previous
import functools

import jax
import jax.numpy as jnp
from jax import lax
from jax.experimental import pallas as pl
from jax.experimental.pallas import tpu as pltpu
from jax.experimental.pallas import tpu_sc as plsc

_B = 16384
_NC = 26
_NN = 13
_V = 100000
_D = 32
_H = 2 * _D
_NIDX = _B * _NC
_PK = 128 // _D

_NW = 32
_BPW = _NIDX // _NW
_CH = 416
_NCH = _BPW // _CH

_CAT_PROWS = _NIDX // _PK
_NUM_PROWS = _B * _NN // _PK
_RB_CAT = 2048
_RB_NUM = _B // _PK

_SQRT2 = 1.4142135623730951

_VP = 100096
_TLB = _VP // 17


def _tpose_body(x_ref, o_ref):
    x = x_ref[0]
    o_ref[:, 0:_D] = jnp.swapaxes(x, 0, 1)


def _transpose_pack(Tn):
    return pl.pallas_call(
        _tpose_body,
        grid=(_NC, _VP // _TLB),
        in_specs=[pl.BlockSpec((1, _D, _TLB), lambda f, j: (f, 0, j))],
        out_specs=pl.BlockSpec((_TLB, 128), lambda f, j: (17 * f + j, 0)),
        out_shape=jax.ShapeDtypeStruct((_NC * _VP, 128), jnp.float32),
        compiler_params=pltpu.CompilerParams(
            dimension_semantics=("parallel", "parallel")),
    )(Tn)


def _sc_gather(t128, idx):
    mesh = plsc.VectorSubcoreMesh(core_axis_name="core", subcore_axis_name="subcore")

    @functools.partial(
        pl.kernel,
        out_type=jax.ShapeDtypeStruct((_CAT_PROWS, 128), jnp.float32),
        mesh=mesh,
        scratch_types=[pltpu.VMEM((_CH,), jnp.int32),
                       pltpu.VMEM((_CH,), jnp.int32),
                       pltpu.VMEM((_CH, 128), jnp.float32),
                       pltpu.VMEM((_CH, 128), jnp.float32),
                       pltpu.VMEM((_CH // _PK, 128), jnp.float32),
                       pltpu.SemaphoreType.DMA,
                       pltpu.SemaphoreType.DMA],
        compiler_params=pltpu.CompilerParams(use_tc_tiling_on_sc=False),
    )
    def k(x_hbm, i_hbm, o_hbm, idx_v0, idx_v1, rows_v0, rows_v1, comp_v,
          sem0, sem1):
        wid = lax.axis_index("subcore") * 2 + lax.axis_index("core")
        base = wid * _BPW

        def start(c, idx_v, rows_v, sem):
            off = base + c * _CH
            pltpu.sync_copy(i_hbm.at[pl.ds(off, _CH)], idx_v)
            pltpu.async_copy(x_hbm.at[idx_v], rows_v, sem)

        def finish(c, idx_v, rows_v, sem):
            pltpu.make_async_copy(x_hbm.at[idx_v], rows_v, sem).wait()

            @pl.loop(0, _CH // _PK)
            def _(r):
                for a in range(_PK):
                    for h in range(_D // 16):
                        comp_v[r, pl.ds(a * _D + h * 16, 16)] = (
                            rows_v[_PK * r + a, pl.ds(h * 16, 16)])

            off = base + c * _CH
            pltpu.sync_copy(comp_v, o_hbm.at[pl.ds(off // _PK, _CH // _PK)])

        start(0, idx_v0, rows_v0, sem0)

        @pl.loop(0, _NCH // 2)
        def _(p):
            start(2 * p + 1, idx_v1, rows_v1, sem1)
            finish(2 * p, idx_v0, rows_v0, sem0)

            @pl.when(p < _NCH // 2 - 1)
            def _():
                start(2 * p + 2, idx_v0, rows_v0, sem0)

            finish(2 * p + 1, idx_v1, rows_v1, sem1)

    return k(t128, idx)


def _mlp_packed(x, w1, b1, w2, b2, g, be, m):
    h = jnp.dot(x, w1, preferred_element_type=jnp.float32) + b1
    h = 0.5 * h * (1.0 + lax.erf(h / _SQRT2))
    y = jnp.dot(h, w2, preferred_element_type=jnp.float32) + b2
    mu = jnp.dot(y, m, preferred_element_type=jnp.float32)
    c = y - mu
    var = jnp.dot(c * c, m, preferred_element_type=jnp.float32)
    return c * lax.rsqrt(var + 1e-5) * g + be


def _cat_body(x_ref, w1_ref, b1_ref, w2_ref, b2_ref, g_ref, be_ref, m_ref,
              o_ref):
    o_ref[...] = _mlp_packed(x_ref[...], w1_ref[...], b1_ref[...], w2_ref[...],
                             b2_ref[...], g_ref[...], be_ref[...], m_ref[...])


def _num_body(v_ref, e_ref, wn_ref, bn_ref, w1_ref, b1_ref, w2_ref, b2_ref,
              g_ref, be_ref, m_ref, o_ref):
    v = jnp.dot(v_ref[...], e_ref[...], preferred_element_type=jnp.float32)
    x = v * wn_ref[0] + bn_ref[0]
    o_ref[...] = _mlp_packed(x, w1_ref[...], b1_ref[...], w2_ref[...],
                             b2_ref[...], g_ref[...], be_ref[...], m_ref[...])


def _full(shape):
    return pl.BlockSpec(shape, lambda i: tuple(0 for _ in shape))


def kernel(cat_inputs, num_inputs, T, Wn, bn, W1, b1, W2, b2, gamma, beta):
    tp = _transpose_pack(T.transpose(0, 2, 1))
    idx = (cat_inputs.T.astype(jnp.int32)
           + (jnp.arange(_NC, dtype=jnp.int32) * _VP)[:, None]).reshape(_NIDX)

    cat_p = _sc_gather(tp, idx)

    eye4 = jnp.eye(_PK, dtype=jnp.float32)
    w1bd = jnp.kron(eye4, W1)
    w2bd = jnp.kron(eye4, W2)
    mavg = jnp.kron(eye4, jnp.full((_D, _D), 1.0 / _D, jnp.float32))
    b1t = jnp.tile(b1, _PK).reshape(1, _PK * _H)
    b2t = jnp.tile(b2, _PK).reshape(1, 128)
    gt = jnp.tile(gamma, _PK).reshape(1, 128)
    bet = jnp.tile(beta, _PK).reshape(1, 128)

    w_specs = [_full((128, _PK * _H)), _full((1, _PK * _H)),
               _full((_PK * _H, 128)), _full((1, 128)), _full((1, 128)),
               _full((1, 128)), _full((128, 128))]

    catm = pl.pallas_call(
        _cat_body,
        grid=(_CAT_PROWS // _RB_CAT,),
        in_specs=[pl.BlockSpec((_RB_CAT, 128), lambda i: (i, 0))] + w_specs,
        out_specs=pl.BlockSpec((_RB_CAT, 128), lambda i: (i, 0)),
        out_shape=jax.ShapeDtypeStruct((_CAT_PROWS, 128), jnp.float32),
    )(cat_p, w1bd, b1t, w2bd, b2t, gt, bet, mavg)

    num4 = num_inputs.T.reshape(_NUM_PROWS, _PK)
    e4 = jnp.kron(eye4, jnp.ones((1, _D), jnp.float32))
    wn_fm = jnp.tile(Wn, (1, _PK)).reshape(_NN, 1, 128)
    bn_fm = jnp.tile(bn, (1, _PK)).reshape(_NN, 1, 128)

    numm = pl.pallas_call(
        _num_body,
        grid=(_NN,),
        in_specs=[pl.BlockSpec((_RB_NUM, _PK), lambda i: (i, 0)),
                  _full((_PK, 128)),
                  pl.BlockSpec((1, 1, 128), lambda i: (i, 0, 0)),
                  pl.BlockSpec((1, 1, 128), lambda i: (i, 0, 0))] + w_specs,
        out_specs=pl.BlockSpec((_RB_NUM, 128), lambda i: (i, 0)),
        out_shape=jax.ShapeDtypeStruct((_NUM_PROWS, 128), jnp.float32),
    )(num4, e4, wn_fm, bn_fm, w1bd, b1t, w2bd, b2t, gt, bet, mavg)

    cat3 = catm.reshape(_NC, _B, _D).transpose(1, 0, 2)
    num3 = numm.reshape(_NN, _B, _D).transpose(1, 0, 2)
    return jnp.concatenate([cat3, num3], axis=1)
